# Initial kernel scaffold; baseline (speedup 1.0000x reference)
import jax
import jax.numpy as jnp
from jax.experimental import pallas as pl


def kernel(x, weight):
    raise NotImplementedError("write your pallas kernel here")


# speedup vs baseline: 1.1580x; 1.1580x over previous
import functools

import jax
import jax.numpy as jnp
from jax import lax
from jax.experimental import pallas as pl
from jax.experimental.pallas import tpu as pltpu
from jax.experimental.pallas import tpu_sc as plsc

D = 128
NUM_CORES = 2
NUM_SUBCORES = 16
NW = NUM_CORES * NUM_SUBCORES
CHUNK = 128


@functools.lru_cache(maxsize=None)
def _make_kernel(n_chunks: int, interpret: bool = False):
    mesh = plsc.VectorSubcoreMesh(
        core_axis_name="c",
        subcore_axis_name="s",
        num_cores=NUM_CORES,
        num_subcores=NUM_SUBCORES,
    )

    @functools.partial(
        pl.kernel,
        out_type=jax.ShapeDtypeStruct((NW, n_chunks, CHUNK, D), jnp.float32),
        mesh=mesh,
        scratch_types=[
            pltpu.VMEM((n_chunks, CHUNK), jnp.int32),
            pltpu.VMEM((CHUNK, D), jnp.float32),
            pltpu.SemaphoreType.DMA,
        ],
        interpret=interpret,
    )
    def emb(x_hbm, w_hbm, out_hbm, idx_v, rows_v, sem):
        wid = lax.axis_index("s") * NUM_CORES + lax.axis_index("c")
        pltpu.sync_copy(x_hbm.at[wid], idx_v)

        def body(j, carry):
            pltpu.async_copy(w_hbm.at[idx_v.at[j]], rows_v, sem).wait()
            pltpu.sync_copy(rows_v, out_hbm.at[wid, j])
            return carry

        lax.fori_loop(0, n_chunks, body, 0)

    return emb


def kernel(x, weight):
    b, f = x.shape
    total = b * f
    n_chunks = total // (NW * CHUNK)
    x_r = x.reshape(NW, n_chunks, CHUNK).astype(jnp.int32)
    out = _make_kernel(n_chunks)(x_r, weight)
    return out.reshape(b, f, D)


# speedup vs baseline: 1.2938x; 1.1173x over previous
import functools

import jax
import jax.numpy as jnp
from jax import lax
from jax.experimental import pallas as pl
from jax.experimental.pallas import tpu as pltpu
from jax.experimental.pallas import tpu_sc as plsc

D = 128
NUM_CORES = 2
NUM_SUBCORES = 16
NW = NUM_CORES * NUM_SUBCORES
CHUNK = 128
NBUF = 4


@functools.lru_cache(maxsize=None)
def _make_kernel(n_chunks: int):
    mesh = plsc.VectorSubcoreMesh(
        core_axis_name="c",
        subcore_axis_name="s",
        num_cores=NUM_CORES,
        num_subcores=NUM_SUBCORES,
    )

    @functools.partial(
        pl.kernel,
        out_type=jax.ShapeDtypeStruct((NW, n_chunks, CHUNK, D), jnp.float32),
        mesh=mesh,
        scratch_types=[
            pltpu.VMEM((n_chunks, CHUNK), jnp.int32),
            pltpu.VMEM((NBUF, CHUNK, D), jnp.float32),
            pltpu.SemaphoreType.DMA,
            pltpu.SemaphoreType.DMA,
        ],
    )
    def emb(x_hbm, w_hbm, out_hbm, idx_v, bufs, gsem, ssem):
        wid = lax.axis_index("s") * NUM_CORES + lax.axis_index("c")
        pltpu.sync_copy(x_hbm.at[wid], idx_v)

        pltpu.async_copy(w_hbm.at[idx_v.at[0]], bufs.at[0], gsem)
        pltpu.async_copy(w_hbm.at[idx_v.at[1]], bufs.at[1], gsem)

        def body(j, carry):
            b = lax.rem(j, NBUF)
            buf = bufs.at[b]
            pltpu.make_async_copy(w_hbm.at[idx_v.at[j]], buf, gsem).wait()
            pltpu.async_copy(buf, out_hbm.at[wid, j], ssem)

            @pl.when(j >= 2)
            def _drain():
                pltpu.make_async_copy(bufs.at[0], out_hbm.at[wid, 0], ssem).wait()

            @pl.when(j + 2 < n_chunks)
            def _next():
                jn = j + 2
                pltpu.async_copy(
                    w_hbm.at[idx_v.at[jn]], bufs.at[lax.rem(jn, NBUF)], gsem
                )

            return carry

        lax.fori_loop(0, n_chunks, body, 0)

        pltpu.make_async_copy(bufs.at[0], out_hbm.at[wid, 0], ssem).wait()
        pltpu.make_async_copy(bufs.at[0], out_hbm.at[wid, 0], ssem).wait()

    return emb


def kernel(x, weight):
    b, f = x.shape
    total = b * f
    n_chunks = total // (NW * CHUNK)
    x_r = x.reshape(NW, n_chunks, CHUNK).astype(jnp.int32)
    out = _make_kernel(n_chunks)(x_r, weight)
    return out.reshape(b, f, D)


# speedup vs baseline: 1.8365x; 1.4194x over previous
import functools

import jax
import jax.numpy as jnp
from jax import lax
from jax.experimental import pallas as pl
from jax.experimental.pallas import tpu as pltpu
from jax.experimental.pallas import tpu_sc as plsc

D = 128
F = 26
NUM_CORES = 2
NUM_SUBCORES = 16
NW = NUM_CORES * NUM_SUBCORES
BCHUNK = 4
NBUF = 4


@functools.lru_cache(maxsize=None)
def _make_kernel(b_total: int):
    b_per_w = b_total // NW
    n_chunks = b_per_w // BCHUNK
    cr = BCHUNK * F
    flat_w = b_per_w * F
    mesh = plsc.VectorSubcoreMesh(
        core_axis_name="c",
        subcore_axis_name="s",
        num_cores=NUM_CORES,
        num_subcores=NUM_SUBCORES,
    )

    @functools.partial(
        pl.kernel,
        out_type=jax.ShapeDtypeStruct((b_total, F, D), jnp.float32),
        mesh=mesh,
        scratch_types=[
            pltpu.VMEM((flat_w,), jnp.int32),
            pltpu.VMEM((NBUF, cr, D), jnp.float32),
            pltpu.SemaphoreType.DMA,
            pltpu.SemaphoreType.DMA,
        ],
    )
    def emb(x_hbm, w_hbm, out_hbm, idx_v, bufs, gsem, ssem):
        wid = lax.axis_index("s") * NUM_CORES + lax.axis_index("c")
        b0 = wid * b_per_w
        for r in range(flat_w // 128):
            pltpu.sync_copy(x_hbm.at[wid, r], idx_v.at[pl.ds(r * 128, 128)])

        def gather(j, buf):
            pltpu.async_copy(w_hbm.at[idx_v.at[pl.ds(j * cr, cr)]], buf, gsem)

        def gather_wait(j, buf):
            pltpu.make_async_copy(
                w_hbm.at[idx_v.at[pl.ds(j * cr, cr)]], buf, gsem
            ).wait()

        def writeback(j, buf):
            for i in range(BCHUNK):
                pltpu.async_copy(
                    buf.at[pl.ds(i * F, F)],
                    out_hbm.at[b0 + j * BCHUNK + i],
                    ssem,
                )

        def drain_one():
            pltpu.make_async_copy(
                bufs.at[0, pl.ds(0, F)], out_hbm.at[0], ssem
            ).wait()

        gather(0, bufs.at[0])
        gather(1, bufs.at[1])

        def body(j, carry):
            buf = bufs.at[lax.rem(j, NBUF)]
            gather_wait(j, buf)
            writeback(j, buf)

            @pl.when(j >= 2)
            def _drain():
                for _ in range(BCHUNK):
                    drain_one()

            @pl.when(j + 2 < n_chunks)
            def _next():
                jn = j + 2
                gather(jn, bufs.at[lax.rem(jn, NBUF)])

            return carry

        lax.fori_loop(0, n_chunks, body, 0)

        for _ in range(2 * BCHUNK):
            drain_one()

    return emb


def kernel(x, weight):
    b, f = x.shape
    x_r = x.reshape(NW, (b // NW * f) // 128, 128).astype(jnp.int32)
    return _make_kernel(b)(x_r, weight)


# speedup vs baseline: 3.8089x; 2.0740x over previous
import functools

import jax
import jax.numpy as jnp
from jax import lax
from jax.experimental import pallas as pl
from jax.experimental.pallas import tpu as pltpu
from jax.experimental.pallas import tpu_sc as plsc

D = 128
NUM_CORES = 2
NUM_SUBCORES = 16
NW = NUM_CORES * NUM_SUBCORES
NBUF = 4


@functools.lru_cache(maxsize=None)
def _make_kernel(b_total: int, f_total: int):
    b_per_w = b_total // NW
    mesh = plsc.VectorSubcoreMesh(
        core_axis_name="c",
        subcore_axis_name="s",
        num_cores=NUM_CORES,
        num_subcores=NUM_SUBCORES,
    )

    @functools.partial(
        pl.kernel,
        out_type=jax.ShapeDtypeStruct((f_total, b_total, D), jnp.float32),
        mesh=mesh,
        scratch_types=[
            pltpu.VMEM((f_total, b_per_w), jnp.int32),
            pltpu.VMEM((NBUF, b_per_w, D), jnp.float32),
            pltpu.SemaphoreType.DMA,
            pltpu.SemaphoreType.DMA,
        ],
    )
    def emb(xt_hbm, w_hbm, out_hbm, idx_v, bufs, gsem, ssem):
        wid = lax.axis_index("s") * NUM_CORES + lax.axis_index("c")
        b0 = wid * b_per_w
        pltpu.sync_copy(xt_hbm.at[:, pl.ds(b0, b_per_w)], idx_v)

        def gather(f, buf):
            pltpu.async_copy(w_hbm.at[idx_v.at[f]], buf, gsem)

        def gather_wait(f, buf):
            pltpu.make_async_copy(w_hbm.at[idx_v.at[f]], buf, gsem).wait()

        def drain_one():
            pltpu.make_async_copy(
                bufs.at[0], out_hbm.at[0, pl.ds(0, b_per_w)], ssem
            ).wait()

        gather(0, bufs.at[0])
        gather(1, bufs.at[1])

        def body(f, carry):
            buf = bufs.at[lax.rem(f, NBUF)]
            gather_wait(f, buf)
            pltpu.async_copy(buf, out_hbm.at[f, pl.ds(b0, b_per_w)], ssem)

            @pl.when(f >= 2)
            def _drain():
                drain_one()

            @pl.when(f + 2 < f_total)
            def _next():
                gather(f + 2, bufs.at[lax.rem(f + 2, NBUF)])

            return carry

        lax.fori_loop(0, f_total, body, 0)

        drain_one()
        drain_one()

    return emb


def kernel(x, weight):
    b, f = x.shape
    xt = jnp.transpose(x).astype(jnp.int32)
    out = _make_kernel(b, f)(xt, weight)
    return jnp.transpose(out, (1, 0, 2))
